# baseline (speedup 1.0000x reference)
import functools

import jax
import jax.numpy as jnp
from jax import lax
from jax.experimental import pallas as pl
from jax.experimental.pallas import tpu as pltpu
from jax.experimental.pallas import tpu_sc as plsc

VOCAB = 100000
EMBED_DIM = 768
BATCH = 4
SEQ = 2048

NUM_CORES = 2
NUM_SUBCORES = 16
NW = NUM_CORES * NUM_SUBCORES
TOTAL = BATCH * SEQ
B_PER_W = TOTAL // NW
CHUNK = 32
SIZES = (16, 16, 32, 32, 32, 32, 32, 32, 16, 16)
assert sum(SIZES) == B_PER_W
OFFS = tuple(sum(SIZES[:i]) for i in range(len(SIZES)))
NCHUNK = len(SIZES)
NBUF = 5
WDELAY = 3


def _make_gather():
    mesh = plsc.VectorSubcoreMesh(core_axis_name="c", subcore_axis_name="s")

    @functools.partial(
        pl.kernel,
        mesh=mesh,
        out_type=jax.ShapeDtypeStruct((BATCH, SEQ, EMBED_DIM), jnp.float32),
        scratch_types=[
            pltpu.VMEM((B_PER_W,), jnp.int32),
        ] + [
            pltpu.VMEM((CHUNK, EMBED_DIM), jnp.float32) for _ in range(NBUF)
        ] + [
            pltpu.SemaphoreType.DMA,
            pltpu.SemaphoreType.DMA,
            pltpu.SemaphoreType.DMA,
        ],
    )
    def k(ids_hbm, table_hbm, out_hbm, idx_v, *rest):
        bufs = rest[:NBUF]
        gsem, wsem, isem = rest[NBUF], rest[NBUF + 1], rest[NBUF + 2]
        wid = lax.axis_index("s") * NUM_CORES + lax.axis_index("c")
        wpb = SEQ // B_PER_W
        row = wid // wpb
        off = (wid % wpb) * B_PER_W
        half = B_PER_W // 2
        head = pltpu.async_copy(
            ids_hbm.at[row, pl.ds(off, half)], idx_v.at[pl.ds(0, half)], isem)
        tail = pltpu.async_copy(
            ids_hbm.at[row, pl.ds(off + half, half)],
            idx_v.at[pl.ds(half, half)], isem)
        g = [None] * NBUF
        w = [None] * NBUF

        def chunk_buf(j, b):
            s = SIZES[j]
            return bufs[b] if s == CHUNK else bufs[b].at[pl.ds(0, s)]

        def gather(j, b):
            return pltpu.async_copy(
                table_hbm.at[idx_v.at[pl.ds(OFFS[j], SIZES[j])]],
                chunk_buf(j, b), gsem)

        nhead = max(i for i in range(NCHUNK + 1) if sum(SIZES[:i]) <= half)
        head.wait()
        for j in range(min(nhead, NBUF, NCHUNK)):
            g[j] = gather(j, j)
        tail.wait()
        for j in range(min(nhead, NBUF, NCHUNK), min(NBUF, NCHUNK)):
            g[j] = gather(j, j)
        for j in range(NCHUNK):
            b = j % NBUF
            g[b].wait()
            w[b] = pltpu.async_copy(
                chunk_buf(j, b),
                out_hbm.at[row, pl.ds(off + OFFS[j], SIZES[j])], wsem)
            jd = j - WDELAY
            if jd >= 0 and jd + NBUF < NCHUNK:
                bd = jd % NBUF
                w[bd].wait()
                w[bd] = None
                g[bd] = gather(jd + NBUF, bd)
        for b in range(NBUF):
            if w[b] is not None:
                w[b].wait()

    return k


_gather = _make_gather()


def kernel(encoder_input_ids, encoder_attention_mask, embed_table):
    ids = encoder_input_ids.astype(jnp.int32)
    hidden = _gather(ids, embed_table)
    return (encoder_input_ids, encoder_attention_mask, hidden)


# speedup vs baseline: 1.0160x; 1.0160x over previous
import functools

import jax
import jax.numpy as jnp
from jax import lax
from jax.experimental import pallas as pl
from jax.experimental.pallas import tpu as pltpu
from jax.experimental.pallas import tpu_sc as plsc

VOCAB = 100000
EMBED_DIM = 768
BATCH = 4
SEQ = 2048

NUM_CORES = 2
NUM_SUBCORES = 16
NW = NUM_CORES * NUM_SUBCORES
TOTAL = BATCH * SEQ
B_PER_W = TOTAL // NW
CHUNK = 32
NCHUNK = B_PER_W // CHUNK
NBUF = 5
WDELAY = 3


def _make_gather():
    mesh = plsc.VectorSubcoreMesh(core_axis_name="c", subcore_axis_name="s")

    @functools.partial(
        pl.kernel,
        mesh=mesh,
        out_type=jax.ShapeDtypeStruct((BATCH, SEQ, EMBED_DIM), jnp.float32),
        scratch_types=[
            pltpu.VMEM((B_PER_W,), jnp.int32),
        ] + [
            pltpu.VMEM((CHUNK, EMBED_DIM), jnp.float32) for _ in range(NBUF)
        ] + [
            pltpu.SemaphoreType.DMA,
            pltpu.SemaphoreType.DMA,
        ],
    )
    def k(ids_hbm, table_hbm, out_hbm, idx_v, *rest):
        bufs = rest[:NBUF]
        gsem, wsem = rest[NBUF], rest[NBUF + 1]
        wid = lax.axis_index("s") * NUM_CORES + lax.axis_index("c")
        wpb = SEQ // B_PER_W
        row = wid // wpb
        off = (wid % wpb) * B_PER_W
        pltpu.sync_copy(ids_hbm.at[row, pl.ds(off, B_PER_W)], idx_v)
        g = [None] * NBUF
        w = [None] * NBUF

        def gather(j, buf):
            return pltpu.async_copy(
                table_hbm.at[idx_v.at[pl.ds(j * CHUNK, CHUNK)]], buf, gsem)

        for j in range(min(NBUF, NCHUNK)):
            g[j] = gather(j, bufs[j])
        for j in range(NCHUNK):
            b = j % NBUF
            g[b].wait()
            w[b] = pltpu.async_copy(
                bufs[b], out_hbm.at[row, pl.ds(off + j * CHUNK, CHUNK)], wsem)
            jd = j - WDELAY
            if jd >= 0 and jd + NBUF < NCHUNK:
                bd = jd % NBUF
                w[bd].wait()
                w[bd] = None
                g[bd] = gather(jd + NBUF, bufs[bd])
        for b in range(NBUF):
            if w[b] is not None:
                w[b].wait()

    return k


_gather = _make_gather()


def kernel(encoder_input_ids, encoder_attention_mask, embed_table):
    ids = encoder_input_ids.astype(jnp.int32)
    hidden = _gather(ids, embed_table)
    return (encoder_input_ids, encoder_attention_mask, hidden)
